# baseline (speedup 1.0000x reference)
import functools

import jax
import jax.numpy as jnp
from jax import lax
from jax.experimental import pallas as pl
from jax.experimental.pallas import tpu as pltpu

_NB = 2


def _fold_bn(gamma, beta, mean, var, eps=1e-5):
    scale = gamma / jnp.sqrt(var + eps)
    return scale, beta - mean * scale


def _cbam_kernel(x_ref,
                 w1_ref, b1_ref,
                 w2_ref, b2_ref, w3_ref, b3_ref,
                 cg1w_ref, cg1b_ref, cg2w_ref, cg2b_ref, sgw_ref, sgb_ref,
                 o_ref, *, H, W):
    HW = H * W
    f32 = jnp.float32
    bf16 = jnp.bfloat16

    wcol = lax.broadcasted_iota(jnp.int32, (1, HW), 1) % W

    def srcmask(off, dtype):
        return ((wcol - off >= 0) & (wcol - off < W)).astype(dtype)

    P = w1_ref.shape[0]
    m3 = {dx: jnp.broadcast_to(srcmask(dx, bf16), (P, HW))
          for dx in (-1, 1)}
    m7 = {dx: srcmask(dx, f32) for dx in (-3, -2, -1, 1, 2, 3)}

    w1 = w1_ref[...]
    y1s = []
    for n in range(_NB):
        xb = x_ref[n].astype(bf16)
        y1 = jnp.dot(w1, xb, preferred_element_type=f32)
        y1s.append(jnp.maximum(y1 + b1_ref[...], 0.0).astype(bf16))
    P = y1s[0].shape[0]

    patchess = []
    for n in range(_NB):
        zpad = jnp.zeros((P, 2 * W), bf16)
        dxbuf = []
        for dx in (-1, 0, 1):
            src = y1s[n] if dx == 0 else y1s[n] * m3[dx]
            dxbuf.append(jnp.concatenate([zpad, src, zpad], axis=1))
        rows = []
        for dy in range(3):
            for dx in range(3):
                s = (dy - 1) * W + (dx - 1)
                rows.append(dxbuf[dx][:, 2 * W + s: 2 * W + s + HW])
        patchess.append(jnp.concatenate(rows, axis=0))

    y2s = []
    for n in range(_NB):
        y2 = jnp.dot(w2_ref[...], patchess[n], preferred_element_type=f32)
        y2s.append(jnp.maximum(y2 + b2_ref[...], 0.0).astype(bf16))

    outs = []
    for n in range(_NB):
        outs.append(jnp.dot(w3_ref[...], y2s[n],
                            preferred_element_type=f32) + b3_ref[...])
    C = outs[0].shape[0]

    vcols = []
    for n in range(_NB):
        out = outs[n]
        vcols.append(jnp.sum(out, axis=1, keepdims=True) * (1.0 / HW))
        vcols.append(jnp.max(out, axis=1, keepdims=True))
    v = jnp.concatenate(vcols, axis=1)
    hmid = jnp.dot(cg1w_ref[...], v, preferred_element_type=f32) + cg1b_ref[...]
    hmid = jnp.maximum(hmid, 0.0)
    yg = jnp.dot(cg2w_ref[...], hmid, preferred_element_type=f32) + cg2b_ref[...]
    atts = [jax.nn.sigmoid(yg[:, 2 * n:2 * n + 1] + yg[:, 2 * n + 1:2 * n + 2])
            for n in range(_NB)]

    gs = [outs[n] * atts[n] for n in range(_NB)]

    sps = []
    for n in range(_NB):
        spmax = jnp.max(gs[n], axis=0, keepdims=True)
        spmean = jnp.sum(gs[n], axis=0, keepdims=True) * (1.0 / C)
        sps.append(jnp.concatenate([spmax, spmean], axis=0))

    sp2 = jnp.concatenate(sps, axis=0)
    R = sp2.shape[0]
    zpad7 = jnp.zeros((R, 4 * W), f32)
    dxbuf7 = []
    for dx in range(-3, 4):
        src = sp2 if dx == 0 else sp2 * m7[dx]
        dxbuf7.append(jnp.concatenate([zpad7, src, zpad7], axis=1))
    rows7 = []
    for dy in range(7):
        for dx in range(7):
            s = (dy - 3) * W + (dx - 3)
            rows7.append(dxbuf7[dx][:, 4 * W + s: 4 * W + s + HW])
    sppat = jnp.concatenate(rows7, axis=0)
    logits = jnp.dot(sgw_ref[...], sppat, preferred_element_type=f32) + sgb_ref[...]
    satt = jax.nn.sigmoid(logits)

    for n in range(_NB):
        o_ref[n] = jnp.maximum(gs[n] * satt[n:n + 1] + x_ref[n], 0.0)


def kernel(x, conv1_w, bn1_g, bn1_b, bn1_m, bn1_v,
           conv2_w, bn2_g, bn2_b, bn2_m, bn2_v,
           conv3_w, bn3_g, bn3_b, bn3_m, bn3_v,
           cg_fc1_w, cg_fc1_b, cg_fc2_w, cg_fc2_b,
           sg_conv_w, sg_bn_g, sg_bn_b, sg_bn_m, sg_bn_v):
    N, Cin, H, W = x.shape
    HW = H * W
    P = conv1_w.shape[0]
    C = conv3_w.shape[0]
    mid = cg_fc1_w.shape[0]
    bf16 = jnp.bfloat16

    s1, t1 = _fold_bn(bn1_g, bn1_b, bn1_m, bn1_v)
    s2, t2 = _fold_bn(bn2_g, bn2_b, bn2_m, bn2_v)
    s3, t3 = _fold_bn(bn3_g, bn3_b, bn3_m, bn3_v)
    ss, ts = _fold_bn(sg_bn_g, sg_bn_b, sg_bn_m, sg_bn_v)

    w1f = (conv1_w.reshape(P, Cin) * s1[:, None]).astype(bf16)
    b1 = t1.reshape(P, 1)
    w2m = jnp.transpose(conv2_w, (0, 2, 3, 1)).reshape(P, 9 * P)
    w2f = (w2m * s2[:, None]).astype(bf16)
    b2 = t2.reshape(P, 1)
    w3f = (conv3_w.reshape(C, P) * s3[:, None]).astype(bf16)
    b3 = t3.reshape(C, 1)

    sgm = (jnp.transpose(sg_conv_w, (0, 2, 3, 1)).reshape(98) *
           ss.reshape(1))
    sgw = jnp.einsum('kc,nm->nkmc', sgm.reshape(49, 2),
                     jnp.eye(_NB, dtype=jnp.float32)).reshape(_NB, 98 * _NB)
    sgb = jnp.broadcast_to(ts.reshape(1, 1), (_NB, 1))

    x_flat = x.reshape(N, Cin, HW)
    inv = lambda i: (0, 0)
    cost = pl.CostEstimate(
        flops=2 * N * HW * (P * Cin + P * 9 * P + C * P) + 8 * N * C * HW,
        transcendentals=N * (C + HW),
        bytes_accessed=N * (Cin + C) * HW * 4,
    )
    out = pl.pallas_call(
        functools.partial(_cbam_kernel, H=H, W=W),
        out_shape=jax.ShapeDtypeStruct((N, C, HW), jnp.float32),
        grid_spec=pltpu.PrefetchScalarGridSpec(
            num_scalar_prefetch=0,
            grid=(N // _NB,),
            in_specs=[
                pl.BlockSpec((_NB, Cin, HW), lambda i: (i, 0, 0)),
                pl.BlockSpec((P, Cin), inv),
                pl.BlockSpec((P, 1), inv),
                pl.BlockSpec((P, 9 * P), inv),
                pl.BlockSpec((P, 1), inv),
                pl.BlockSpec((C, P), inv),
                pl.BlockSpec((C, 1), inv),
                pl.BlockSpec((mid, Cin), inv),
                pl.BlockSpec((mid, 1), inv),
                pl.BlockSpec((C, mid), inv),
                pl.BlockSpec((C, 1), inv),
                pl.BlockSpec((_NB, 49 * 2 * _NB), inv),
                pl.BlockSpec((_NB, 1), inv),
            ],
            out_specs=pl.BlockSpec((_NB, C, HW), lambda i: (i, 0, 0)),
        ),
        compiler_params=pltpu.CompilerParams(
            dimension_semantics=("parallel",),
            vmem_limit_bytes=48 << 20,
        ),
        cost_estimate=cost,
    )(x_flat, w1f, b1, w2f, b2, w3f, b3,
      cg_fc1_w, cg_fc1_b.reshape(mid, 1), cg_fc2_w, cg_fc2_b.reshape(C, 1),
      sgw, sgb)
    return out.reshape(N, C, H, W)


# speedup vs baseline: 1.0090x; 1.0090x over previous
import functools

import jax
import jax.numpy as jnp
from jax import lax
from jax.experimental import pallas as pl
from jax.experimental.pallas import tpu as pltpu

_NB = 2


def _fold_bn(gamma, beta, mean, var, eps=1e-5):
    scale = gamma / jnp.sqrt(var + eps)
    return scale, beta - mean * scale


def _cbam_kernel(x_ref,
                 w1_ref, b1_ref,
                 w2_ref, b2_ref, w3_ref, b3_ref,
                 cg1w_ref, cg1b_ref, cg2w_ref, cg2b_ref, sgw_ref, sgb_ref,
                 o_ref, *, H, W):
    HW = H * W
    f32 = jnp.float32
    bf16 = jnp.bfloat16

    wcol = lax.broadcasted_iota(jnp.int32, (1, HW), 1) % W

    def srcmask(off, dtype):
        return ((wcol - off >= 0) & (wcol - off < W)).astype(dtype)

    P = w1_ref.shape[0]
    m3 = {dx: jnp.broadcast_to(srcmask(dx, bf16), (P, HW))
          for dx in (-1, 1)}
    m7 = {dx: srcmask(dx, f32) for dx in (-3, -2, -1, 1, 2, 3)}

    w1 = w1_ref[...]
    y1s = []
    for n in range(_NB):
        xb = x_ref[n].astype(bf16)
        y1 = jnp.dot(w1, xb, preferred_element_type=f32)
        y1s.append(jnp.maximum(y1 + b1_ref[...], 0.0).astype(bf16))
    P = y1s[0].shape[0]

    patchess = []
    for n in range(_NB):
        zpad = jnp.zeros((P, 2 * W), bf16)
        y1p = jnp.concatenate([zpad, y1s[n], zpad], axis=1)
        rows = []
        for dy in range(3):
            for dx in range(3):
                s = (dy - 1) * W + (dx - 1)
                sl = y1p[:, 2 * W + s: 2 * W + s + HW]
                if dx != 1:
                    sl = sl * m3[1 - dx]
                rows.append(sl)
        patchess.append(jnp.concatenate(rows, axis=0))

    y2s = []
    for n in range(_NB):
        y2 = jnp.dot(w2_ref[...], patchess[n], preferred_element_type=f32)
        y2s.append(jnp.maximum(y2 + b2_ref[...], 0.0).astype(bf16))

    outs = []
    for n in range(_NB):
        outs.append(jnp.dot(w3_ref[...], y2s[n],
                            preferred_element_type=f32) + b3_ref[...])
    C = outs[0].shape[0]

    vcols = []
    for n in range(_NB):
        out = outs[n]
        vcols.append(jnp.sum(out, axis=1, keepdims=True) * (1.0 / HW))
        vcols.append(jnp.max(out, axis=1, keepdims=True))
    v = jnp.concatenate(vcols, axis=1)
    hmid = jnp.dot(cg1w_ref[...], v, preferred_element_type=f32) + cg1b_ref[...]
    hmid = jnp.maximum(hmid, 0.0)
    yg = jnp.dot(cg2w_ref[...], hmid, preferred_element_type=f32) + cg2b_ref[...]
    atts = [jax.nn.sigmoid(yg[:, 2 * n:2 * n + 1] + yg[:, 2 * n + 1:2 * n + 2])
            for n in range(_NB)]

    gs = [outs[n] * atts[n] for n in range(_NB)]

    sps = []
    for n in range(_NB):
        spmax = jnp.max(gs[n], axis=0, keepdims=True)
        spmean = jnp.sum(gs[n], axis=0, keepdims=True) * (1.0 / C)
        sps.append(jnp.concatenate([spmax, spmean], axis=0))

    sp2 = jnp.concatenate(sps, axis=0)
    R = sp2.shape[0]
    zpad7 = jnp.zeros((R, 4 * W), f32)
    dxbuf7 = []
    for dx in range(-3, 4):
        src = sp2 if dx == 0 else sp2 * m7[dx]
        dxbuf7.append(jnp.concatenate([zpad7, src, zpad7], axis=1))
    rows7 = []
    for dy in range(7):
        for dx in range(7):
            s = (dy - 3) * W + (dx - 3)
            rows7.append(dxbuf7[dx][:, 4 * W + s: 4 * W + s + HW])
    sppat = jnp.concatenate(rows7, axis=0)
    logits = jnp.dot(sgw_ref[...], sppat, preferred_element_type=f32) + sgb_ref[...]
    satt = jax.nn.sigmoid(logits)

    for n in range(_NB):
        o_ref[n] = jnp.maximum(gs[n] * satt[n:n + 1] + x_ref[n], 0.0)


def kernel(x, conv1_w, bn1_g, bn1_b, bn1_m, bn1_v,
           conv2_w, bn2_g, bn2_b, bn2_m, bn2_v,
           conv3_w, bn3_g, bn3_b, bn3_m, bn3_v,
           cg_fc1_w, cg_fc1_b, cg_fc2_w, cg_fc2_b,
           sg_conv_w, sg_bn_g, sg_bn_b, sg_bn_m, sg_bn_v):
    N, Cin, H, W = x.shape
    HW = H * W
    P = conv1_w.shape[0]
    C = conv3_w.shape[0]
    mid = cg_fc1_w.shape[0]
    bf16 = jnp.bfloat16

    s1, t1 = _fold_bn(bn1_g, bn1_b, bn1_m, bn1_v)
    s2, t2 = _fold_bn(bn2_g, bn2_b, bn2_m, bn2_v)
    s3, t3 = _fold_bn(bn3_g, bn3_b, bn3_m, bn3_v)
    ss, ts = _fold_bn(sg_bn_g, sg_bn_b, sg_bn_m, sg_bn_v)

    w1f = (conv1_w.reshape(P, Cin) * s1[:, None]).astype(bf16)
    b1 = t1.reshape(P, 1)
    w2m = jnp.transpose(conv2_w, (0, 2, 3, 1)).reshape(P, 9 * P)
    w2f = (w2m * s2[:, None]).astype(bf16)
    b2 = t2.reshape(P, 1)
    w3f = (conv3_w.reshape(C, P) * s3[:, None]).astype(bf16)
    b3 = t3.reshape(C, 1)

    sgm = (jnp.transpose(sg_conv_w, (0, 2, 3, 1)).reshape(98) *
           ss.reshape(1))
    sgw = jnp.einsum('kc,nm->nkmc', sgm.reshape(49, 2),
                     jnp.eye(_NB, dtype=jnp.float32)).reshape(_NB, 98 * _NB)
    sgb = jnp.broadcast_to(ts.reshape(1, 1), (_NB, 1))

    x_flat = x.reshape(N, Cin, HW)
    inv = lambda i: (0, 0)
    cost = pl.CostEstimate(
        flops=2 * N * HW * (P * Cin + P * 9 * P + C * P) + 8 * N * C * HW,
        transcendentals=N * (C + HW),
        bytes_accessed=N * (Cin + C) * HW * 4,
    )
    out = pl.pallas_call(
        functools.partial(_cbam_kernel, H=H, W=W),
        out_shape=jax.ShapeDtypeStruct((N, C, HW), jnp.float32),
        grid_spec=pltpu.PrefetchScalarGridSpec(
            num_scalar_prefetch=0,
            grid=(N // _NB,),
            in_specs=[
                pl.BlockSpec((_NB, Cin, HW), lambda i: (i, 0, 0)),
                pl.BlockSpec((P, Cin), inv),
                pl.BlockSpec((P, 1), inv),
                pl.BlockSpec((P, 9 * P), inv),
                pl.BlockSpec((P, 1), inv),
                pl.BlockSpec((C, P), inv),
                pl.BlockSpec((C, 1), inv),
                pl.BlockSpec((mid, Cin), inv),
                pl.BlockSpec((mid, 1), inv),
                pl.BlockSpec((C, mid), inv),
                pl.BlockSpec((C, 1), inv),
                pl.BlockSpec((_NB, 49 * 2 * _NB), inv),
                pl.BlockSpec((_NB, 1), inv),
            ],
            out_specs=pl.BlockSpec((_NB, C, HW), lambda i: (i, 0, 0)),
        ),
        compiler_params=pltpu.CompilerParams(
            dimension_semantics=("parallel",),
            vmem_limit_bytes=48 << 20,
        ),
        cost_estimate=cost,
    )(x_flat, w1f, b1, w2f, b2, w3f, b3,
      cg_fc1_w, cg_fc1_b.reshape(mid, 1), cg_fc2_w, cg_fc2_b.reshape(C, 1),
      sgw, sgb)
    return out.reshape(N, C, H, W)
